# baseline (speedup 1.0000x reference)
import functools

import jax
import jax.numpy as jnp
from jax import lax
from jax.experimental import pallas as pl
from jax.experimental.pallas import tpu as pltpu
from jax.experimental.pallas import tpu_sc as plsc

VOCAB = 1000000
DIM = 64
BATCH = 4096
HIST = 50

NC = 2
NS = 16
NW = NC * NS
BAGS_W = BATCH // NW
ROWS_W = BAGS_W * HIST
CB = 4
CR = CB * HIST
NCHUNK = BAGS_W // CB
SPLIT0 = 104
SPLIT1 = CR - SPLIT0
NBUF = 2
QW = 2 * DIM


def _ebag_body(idx_hbm, sc_hbm, w_hbm, out_hbm,
               idx_v, idxq_v, sc_v, rows_v, out_v, sem0, sem1):
    wid = lax.axis_index("s") * NC + lax.axis_index("c")
    base_row = wid * ROWS_W

    pltpu.sync_copy(idx_hbm.at[pl.ds(base_row, ROWS_W)],
                    idx_v.at[pl.ds(0, ROWS_W)])
    pltpu.sync_copy(sc_hbm.at[pl.ds(base_row, ROWS_W)],
                    sc_v.at[pl.ds(0, ROWS_W)])

    @pl.loop(0, ROWS_W // 16)
    def q_loop(i):
        idxq_v[pl.ds(16 * i, 16)] = idx_v[pl.ds(16 * i, 16)] >> 2

    sems = (sem0, sem1)

    def gather_descs(c, b):
        off = c * CR
        d0 = pltpu.make_async_copy(
            w_hbm.at[idxq_v.at[pl.ds(off, SPLIT0)]],
            rows_v.at[b, pl.ds(0, SPLIT0)],
            sems[b])
        d1 = pltpu.make_async_copy(
            w_hbm.at[idxq_v.at[pl.ds(off + SPLIT0, SPLIT1)]],
            rows_v.at[b, pl.ds(SPLIT0, SPLIT1)],
            sems[b])
        return d0, d1

    for b in range(NBUF):
        for d in gather_descs(b, b):
            d.start()

    iota = lax.iota(jnp.int32, 16)
    zero = jnp.zeros((16,), jnp.float32)

    @pl.loop(0, NCHUNK, step=NBUF)
    def chunk_loop(c0):
        for b in range(NBUF):
            c = c0 + b
            for d in gather_descs(c, b):
                d.wait()

            @pl.loop(0, CB)
            def bag_loop(k, _b=b):
                bag = c * CB + k
                sc_off = bag * HIST
                r0 = k * HIST
                svs = [sc_v[pl.ds(sc_off + 16 * g, 16)] for g in range(4)]
                subs = [(idx_v[pl.ds(sc_off + 16 * g, 16)] & 3) << 5
                        for g in range(4)]
                accs = [zero, zero, zero, zero]
                himask = jnp.full((16,), -65536, jnp.int32)
                for l in range(HIST):
                    s = svs[l // 16][l % 16]
                    sub = subs[l // 16][l % 16]
                    w0 = rows_v[_b, r0 + l, pl.ds(sub, 16)]
                    w1 = rows_v[_b, r0 + l, pl.ds(sub + 16, 16)]
                    e0 = plsc.bitcast(w0 << 16, jnp.float32)
                    o0 = plsc.bitcast(w0 & himask, jnp.float32)
                    e1 = plsc.bitcast(w1 << 16, jnp.float32)
                    o1 = plsc.bitcast(w1 & himask, jnp.float32)
                    accs = [accs[0] + e0 * s, accs[1] + o0 * s,
                            accs[2] + e1 * s, accs[3] + o1 * s]

                bagv = jnp.full((16,), bag, jnp.int32)
                plsc.store_scatter(out_v, [bagv, 2 * iota], accs[0])
                plsc.store_scatter(out_v, [bagv, 2 * iota + 1], accs[1])
                plsc.store_scatter(out_v, [bagv, 32 + 2 * iota], accs[2])
                plsc.store_scatter(out_v, [bagv, 33 + 2 * iota], accs[3])

            @pl.when(c + NBUF < NCHUNK)
            def _():
                for d in gather_descs(c + NBUF, b):
                    d.start()

    pltpu.sync_copy(out_v, out_hbm.at[pl.ds(wid * BAGS_W, BAGS_W), :])


@jax.jit
def _ebag(idx_flat, sc_flat, weight):
    mesh = plsc.VectorSubcoreMesh(core_axis_name="c", subcore_axis_name="s")
    f = pl.kernel(
        _ebag_body,
        out_type=jax.ShapeDtypeStruct((BATCH, DIM), jnp.float32),
        mesh=mesh,
        compiler_params=pltpu.CompilerParams(needs_layout_passes=False),
        scratch_types=[
            pltpu.VMEM((ROWS_W + 16,), jnp.int32),
            pltpu.VMEM((ROWS_W,), jnp.int32),
            pltpu.VMEM((ROWS_W + 16,), jnp.float32),
            pltpu.VMEM((NBUF, CR, QW), jnp.int32),
            pltpu.VMEM((BAGS_W, DIM), jnp.float32),
            pltpu.SemaphoreType.DMA,
            pltpu.SemaphoreType.DMA,
        ],
    )
    return f(idx_flat, sc_flat, weight)


def kernel(indices, scores, weight):
    w128 = lax.bitcast_convert_type(
        weight.reshape(VOCAB // 4, 2 * DIM, 2), jnp.int32)
    out = _ebag(indices.reshape(-1), scores.reshape(-1), w128)
    return out.astype(jnp.bfloat16)


# speedup vs baseline: 47.4462x; 47.4462x over previous
import functools

import jax
import jax.numpy as jnp
from jax import lax
from jax.experimental import pallas as pl
from jax.experimental.pallas import tpu as pltpu
from jax.experimental.pallas import tpu_sc as plsc

VOCAB = 1000000
DIM = 64
BATCH = 4096
HIST = 50

NC = 2
NS = 16
NW = NC * NS
BAGS_W = BATCH // NW
ROWS_W = BAGS_W * HIST
CB = 4
CR = CB * HIST
NCHUNK = BAGS_W // CB
SPLIT0 = 104
SPLIT1 = CR - SPLIT0
NBUF = 2
QW = 2 * DIM


def _ebag_body(idx_hbm, sc_hbm, w_hbm, out_hbm,
               idx_v, idxq_v, sc_v, rows_v, out_v, sem0, sem1):
    wid = lax.axis_index("s") * NC + lax.axis_index("c")
    base_row = wid * ROWS_W

    pltpu.sync_copy(idx_hbm.at[pl.ds(base_row, ROWS_W)],
                    idx_v.at[pl.ds(0, ROWS_W)])
    pltpu.sync_copy(sc_hbm.at[pl.ds(base_row, ROWS_W)],
                    sc_v.at[pl.ds(0, ROWS_W)])

    half = jnp.full((16,), VOCAB // 2, jnp.int32)

    @pl.loop(0, ROWS_W // 16)
    def q_loop(i):
        v = idx_v[pl.ds(16 * i, 16)]
        idxq_v[pl.ds(16 * i, 16)] = jnp.where(v >= half, v - half, v) >> 1

    sems = (sem0, sem1)

    def gather_descs(c, b):
        off = c * CR
        d0 = pltpu.make_async_copy(
            w_hbm.at[idxq_v.at[pl.ds(off, SPLIT0)]],
            rows_v.at[b, pl.ds(0, SPLIT0)],
            sems[b])
        d1 = pltpu.make_async_copy(
            w_hbm.at[idxq_v.at[pl.ds(off + SPLIT0, SPLIT1)]],
            rows_v.at[b, pl.ds(SPLIT0, SPLIT1)],
            sems[b])
        return d0, d1

    for b in range(NBUF):
        for d in gather_descs(b, b):
            d.start()

    iota = lax.iota(jnp.int32, 16)
    zero = jnp.zeros((16,), jnp.float32)

    @pl.loop(0, NCHUNK, step=NBUF)
    def chunk_loop(c0):
        for b in range(NBUF):
            c = c0 + b
            for d in gather_descs(c, b):
                d.wait()

            @pl.loop(0, CB)
            def bag_loop(k, _b=b):
                bag = c * CB + k
                sc_off = bag * HIST
                r0 = k * HIST
                svs = [sc_v[pl.ds(sc_off + 16 * g, 16)] for g in range(4)]
                svi = [idx_v[pl.ds(sc_off + 16 * g, 16)] for g in range(4)]
                subs = [jnp.where(v >= half, 64, 0) for v in svi]
                shs = [16 - ((v & 1) << 4) for v in svi]
                accs = [zero, zero, zero, zero]
                himask = jnp.full((16,), -65536, jnp.int32)
                for l in range(HIST):
                    s = svs[l // 16][l % 16]
                    sub = subs[l // 16][l % 16]
                    sh = shs[l // 16][l % 16]
                    for g in range(4):
                        w = rows_v[_b, r0 + l, pl.ds(sub + 16 * g, 16)]
                        f = plsc.bitcast((w << sh) & himask, jnp.float32)
                        accs[g] = accs[g] + f * s

                for g in range(4):
                    out_v[bag, pl.ds(16 * g, 16)] = accs[g]

            @pl.when(c + NBUF < NCHUNK)
            def _():
                for d in gather_descs(c + NBUF, b):
                    d.start()

    pltpu.sync_copy(out_v, out_hbm.at[pl.ds(wid * BAGS_W, BAGS_W), :])


@jax.jit
def _ebag(idx_flat, sc_flat, weight):
    mesh = plsc.VectorSubcoreMesh(core_axis_name="c", subcore_axis_name="s")
    f = pl.kernel(
        _ebag_body,
        out_type=jax.ShapeDtypeStruct((BATCH, DIM), jnp.float32),
        mesh=mesh,
        compiler_params=pltpu.CompilerParams(needs_layout_passes=False),
        scratch_types=[
            pltpu.VMEM((ROWS_W + 16,), jnp.int32),
            pltpu.VMEM((ROWS_W,), jnp.int32),
            pltpu.VMEM((ROWS_W + 16,), jnp.float32),
            pltpu.VMEM((NBUF, CR, QW), jnp.int32),
            pltpu.VMEM((BAGS_W, DIM), jnp.float32),
            pltpu.SemaphoreType.DMA,
            pltpu.SemaphoreType.DMA,
        ],
    )
    return f(idx_flat, sc_flat, weight)


PACK_BQ = 2000


def _pack_body(xlo_ref, xhi_ref, o_ref):
    o_ref[:, 0:DIM] = pltpu.bitcast(xlo_ref[...], jnp.int32)
    o_ref[:, DIM:2 * DIM] = pltpu.bitcast(xhi_ref[...], jnp.int32)


def _pack_table(weight):
    nblk = VOCAB // (4 * PACK_BQ)
    return pl.pallas_call(
        _pack_body,
        out_shape=jax.ShapeDtypeStruct((VOCAB // 4, 2 * DIM), jnp.int32),
        grid=(nblk,),
        in_specs=[
            pl.BlockSpec((2 * PACK_BQ, DIM), lambda g: (g, 0)),
            pl.BlockSpec((2 * PACK_BQ, DIM), lambda g, n=nblk: (n + g, 0)),
        ],
        out_specs=pl.BlockSpec((PACK_BQ, 2 * DIM), lambda g: (g, 0)),
    )(weight, weight)


def kernel(indices, scores, weight):
    w128 = _pack_table(weight)
    out = _ebag(indices.reshape(-1), scores.reshape(-1), w128)
    return out.astype(jnp.bfloat16)
